# Initial kernel scaffold; baseline (speedup 1.0000x reference)
import jax
import jax.numpy as jnp
from jax.experimental import pallas as pl


def kernel(edge_attr, edge_index, num_nodes, W_v_in, b_v_in, W_q_in, b_q_in, W_k_in, b_k_in, W_v_out, b_v_out, W_q_out, b_q_out, W_k_out, b_k_out, W_o, b_o):
    raise NotImplementedError("write your pallas kernel here")


# speedup vs baseline: 7.7870x; 7.7870x over previous
import functools
import math

import jax
import jax.numpy as jnp
from jax import lax
from jax.experimental import pallas as pl
from jax.experimental.pallas import tpu as pltpu
from jax.experimental.pallas import tpu_sc as plsc

N_NODES = 10000
E = 320000
D_HEAD = 32
INV_SCALE = 1.0 / math.sqrt(D_HEAD)

NPAD = 10240
DUMMY = 10200
TBL = 4 * NPAD
SLICE = TBL // 16
NT = 16
EPAD = 327680
EPT = EPAD // NT
C = 2560

BM1 = 1280
BM2 = 640


def _p1_body(x_ref, wqk_ref, bqk_ref, rep_ref, aT_ref):
    x = x_ref[...]
    qk = jnp.dot(x, wqk_ref[...], preferred_element_type=jnp.float32) + bqk_ref[...]
    r_in = qk[:, 0:128] * qk[:, 128:256]
    r_out = qk[:, 256:384] * qk[:, 384:512]
    r_all = jnp.concatenate([r_in, r_out], axis=1)
    aT = lax.dot_general(rep_ref[...], r_all, (((1,), (1,)), ((), ())),
                         preferred_element_type=jnp.float32)
    aT_ref[...] = aT * INV_SCALE


def _p2_body(x_ref, wv_ref, bv_ref, rep_ref, wTi_ref, wTo_ref, wo_ref, bo_ref,
             out_ref):
    x = x_ref[...]
    v = jnp.dot(x, wv_ref[...], preferred_element_type=jnp.float32) + bv_ref[...]
    rep4 = rep_ref[...]
    wbc_in = lax.dot_general(wTi_ref[...], rep4, (((0,), (0,)), ((), ())),
                             preferred_element_type=jnp.float32)
    wbc_out = lax.dot_general(wTo_ref[...], rep4, (((0,), (0,)), ((), ())),
                              preferred_element_type=jnp.float32)
    o = v * jnp.concatenate([wbc_in, wbc_out], axis=1)
    up = jnp.roll(o, -1, axis=0)
    dn = jnp.roll(o, 1, axis=0)
    row = lax.broadcasted_iota(jnp.int32, (BM2, 1), 0)
    osw = jnp.where((row % 2) == 0, up, dn)
    big = jnp.concatenate(
        [o[:, 0:128], osw[:, 0:128], o[:, 128:256], osw[:, 128:256]], axis=1)
    out_ref[...] = (
        jnp.dot(big, wo_ref[...], preferred_element_type=jnp.float32) + bo_ref[...])


def _softmax_sc_body(aT_hbm, seg_hbm, wTi_hbm, wTo_hbm,
                     table_v, seg_v, a_v, w_v, tmp_v, acc_v, sp_tabs, sp_fin):
    c = lax.axis_index("c")
    s = lax.axis_index("s")
    seg_off0 = (1 - c) * EPAD
    arow0 = c * 4

    zero16 = jnp.zeros((16,), jnp.float32)

    def zb(i, _):
        table_v[pl.ds(i * 16, 16)] = zero16
        return 0
    lax.fori_loop(0, TBL // 16, zb, 0)

    def load_chunk(k):
        base = pl.multiple_of(s * EPT + k * C, 128)
        pltpu.sync_copy(seg_hbm.at[pl.ds(pl.multiple_of(seg_off0 + base, 128), C)],
                        seg_v)
        pltpu.sync_copy(aT_hbm.at[:, pl.ds(base, C)], a_v)
        return base

    for k in range(EPT // C):
        load_chunk(k)

        def sbody(i, _):
            sg = seg_v[pl.ds(i * 16, 16)]
            for h in range(4):
                e = jnp.exp(a_v[arow0 + h, pl.ds(i * 16, 16)])
                plsc.addupdate_scatter(table_v, [sg + (h * NPAD)], e)
            return 0
        lax.fori_loop(0, C // 16, sbody, 0)

    pltpu.sync_copy(table_v, sp_tabs.at[pl.ds(pl.multiple_of(s * TBL, 128), TBL)])
    plsc.subcore_barrier()
    pltpu.sync_copy(sp_tabs.at[pl.ds(pl.multiple_of(s * SLICE, 128), SLICE)],
                    acc_v)
    for t in range(1, NT):
        pltpu.sync_copy(
            sp_tabs.at[pl.ds(pl.multiple_of(t * TBL + s * SLICE, 128), SLICE)],
            tmp_v)

        def mb(i, _):
            acc_v[pl.ds(i * 16, 16)] = (
                acc_v[pl.ds(i * 16, 16)] + tmp_v[pl.ds(i * 16, 16)])
            return 0
        lax.fori_loop(0, SLICE // 16, mb, 0)
    pltpu.sync_copy(acc_v, sp_fin.at[pl.ds(pl.multiple_of(s * SLICE, 128), SLICE)])
    plsc.subcore_barrier()
    pltpu.sync_copy(sp_fin, table_v)

    for k in range(EPT // C):
        base = load_chunk(k)

        def gbody(i, _):
            sg = seg_v[pl.ds(i * 16, 16)]
            for h in range(4):
                e = jnp.exp(a_v[arow0 + h, pl.ds(i * 16, 16)])
                ssum = plsc.load_gather(table_v, [sg + (h * NPAD)])
                w_v[h, pl.ds(i * 16, 16)] = e / (ssum + 1e-16)
            return 0
        lax.fori_loop(0, C // 16, gbody, 0)

        @pl.when(c == 0)
        def _():
            pltpu.sync_copy(w_v, wTi_hbm.at[:, pl.ds(base, C)])

        @pl.when(c == 1)
        def _():
            pltpu.sync_copy(w_v, wTo_hbm.at[:, pl.ds(base, C)])


@functools.lru_cache(maxsize=1)
def _make_softmax_sc():
    mesh = plsc.VectorSubcoreMesh(core_axis_name="c", subcore_axis_name="s")
    return pl.kernel(
        _softmax_sc_body,
        mesh=mesh,
        compiler_params=pltpu.CompilerParams(needs_layout_passes=False),
        out_type=[jax.ShapeDtypeStruct((4, EPAD), jnp.float32),
                  jax.ShapeDtypeStruct((4, EPAD), jnp.float32)],
        scratch_types=[
            pltpu.VMEM((TBL,), jnp.float32),
            pltpu.VMEM((C,), jnp.int32),
            pltpu.VMEM((8, C), jnp.float32),
            pltpu.VMEM((4, C), jnp.float32),
            pltpu.VMEM((SLICE,), jnp.float32),
            pltpu.VMEM((SLICE,), jnp.float32),
            pltpu.VMEM_SHARED((NT * TBL,), jnp.float32),
            pltpu.VMEM_SHARED((TBL,), jnp.float32),
        ],
    )


def _rep8():
    return (jnp.arange(256)[None, :] // 32 == jnp.arange(8)[:, None]).astype(jnp.float32)


def kernel(edge_attr, edge_index, num_nodes,
           W_v_in, b_v_in, W_q_in, b_q_in, W_k_in, b_k_in,
           W_v_out, b_v_out, W_q_out, b_q_out, W_k_out, b_k_out, W_o, b_o):
    del num_nodes
    rep = _rep8()
    rep4 = rep[:4, :128]
    wqk = jnp.concatenate([W_q_in, W_k_in, W_q_out, W_k_out], axis=1)
    bqk = jnp.concatenate([b_q_in, b_k_in, b_q_out, b_k_out])[None, :]
    wv = jnp.concatenate([W_v_in, W_v_out], axis=1)
    bv = jnp.concatenate([b_v_in, b_v_out])[None, :]

    nblk = E // BM1
    aT = pl.pallas_call(
        _p1_body,
        grid=(EPAD // BM1,),
        in_specs=[
            pl.BlockSpec((BM1, 128), lambda i: (jnp.minimum(i, nblk - 1), 0)),
            pl.BlockSpec((128, 512), lambda i: (0, 0)),
            pl.BlockSpec((1, 512), lambda i: (0, 0)),
            pl.BlockSpec((8, 256), lambda i: (0, 0)),
        ],
        out_specs=pl.BlockSpec((8, BM1), lambda i: (0, i)),
        out_shape=jax.ShapeDtypeStruct((8, EPAD), jnp.float32),
    )(edge_attr, wqk, bqk, rep)

    pad = jnp.full((EPAD - E,), DUMMY, jnp.int32)
    seg_all = jnp.concatenate(
        [edge_index[0], pad, edge_index[1], pad])

    wT_in, wT_out = _make_softmax_sc()(aT, seg_all)

    out = pl.pallas_call(
        _p2_body,
        grid=(E // BM2,),
        in_specs=[
            pl.BlockSpec((BM2, 128), lambda i: (i, 0)),
            pl.BlockSpec((128, 256), lambda i: (0, 0)),
            pl.BlockSpec((1, 256), lambda i: (0, 0)),
            pl.BlockSpec((4, 128), lambda i: (0, 0)),
            pl.BlockSpec((4, BM2), lambda i: (0, i)),
            pl.BlockSpec((4, BM2), lambda i: (0, i)),
            pl.BlockSpec((512, 128), lambda i: (0, 0)),
            pl.BlockSpec((1, 128), lambda i: (0, 0)),
        ],
        out_specs=pl.BlockSpec((BM2, 128), lambda i: (i, 0)),
        out_shape=jax.ShapeDtypeStruct((E, 128), jnp.float32),
    )(edge_attr, wv, bv, rep4, wT_in, wT_out, W_o, b_o[None, :])
    return out


# speedup vs baseline: 7.9993x; 1.0273x over previous
import functools
import math

import jax
import jax.numpy as jnp
from jax import lax
from jax.experimental import pallas as pl
from jax.experimental.pallas import tpu as pltpu
from jax.experimental.pallas import tpu_sc as plsc

N_NODES = 10000
E = 320000
D_HEAD = 32
INV_SCALE = 1.0 / math.sqrt(D_HEAD)

NPAD = 10240
DUMMY = 10200
TBL = 4 * NPAD
SLICE = TBL // 16
NT = 16
EPAD = 327680
EPT = EPAD // NT
C = 2560

BM1 = 1280
BM2 = 640


def _p1_body(x_ref, wqk_ref, bqk_ref, rep_ref, aT_ref):
    x = x_ref[...]
    qk = jnp.dot(x, wqk_ref[...], preferred_element_type=jnp.float32) + bqk_ref[...]
    r_in = qk[:, 0:128] * qk[:, 128:256]
    r_out = qk[:, 256:384] * qk[:, 384:512]
    r_all = jnp.concatenate([r_in, r_out], axis=1)
    aT = lax.dot_general(rep_ref[...], r_all, (((1,), (1,)), ((), ())),
                         preferred_element_type=jnp.float32)
    aT_ref[...] = aT * INV_SCALE


def _p2_body(x_ref, wv_ref, bv_ref, rep_ref, wTi_ref, wTo_ref, w13_ref, w24_ref,
             bo_ref, out_ref):
    x = x_ref[...]
    v = jnp.dot(x, wv_ref[...], preferred_element_type=jnp.float32) + bv_ref[...]
    rep4 = rep_ref[...]
    wbc_in = lax.dot_general(wTi_ref[...], rep4, (((0,), (0,)), ((), ())),
                             preferred_element_type=jnp.float32)
    wbc_out = lax.dot_general(wTo_ref[...], rep4, (((0,), (0,)), ((), ())),
                              preferred_element_type=jnp.float32)
    o = v * jnp.concatenate([wbc_in, wbc_out], axis=1)
    z = jnp.dot(o, w24_ref[...], preferred_element_type=jnp.float32)
    up = jnp.roll(z, -1, axis=0)
    dn = jnp.roll(z, 1, axis=0)
    row = lax.broadcasted_iota(jnp.int32, (BM2, 1), 0)
    zsw = jnp.where((row % 2) == 0, up, dn)
    out_ref[...] = (
        jnp.dot(o, w13_ref[...], preferred_element_type=jnp.float32)
        + zsw + bo_ref[...])


def _softmax_sc_body(aT_hbm, seg_hbm, wTi_hbm, wTo_hbm,
                     table_v, seg_v, a_v, w_v, tmp_v, acc_v, sp_tabs, sp_fin):
    c = lax.axis_index("c")
    s = lax.axis_index("s")
    seg_off0 = (1 - c) * EPAD
    arow0 = c * 4

    zero16 = jnp.zeros((16,), jnp.float32)

    def zb(i, _):
        table_v[pl.ds(i * 16, 16)] = zero16
        return 0
    lax.fori_loop(0, TBL // 16, zb, 0)

    def load_chunk(k):
        base = pl.multiple_of(s * EPT + k * C, 128)
        pltpu.sync_copy(seg_hbm.at[pl.ds(pl.multiple_of(seg_off0 + base, 128), C)],
                        seg_v)
        pltpu.sync_copy(aT_hbm.at[:, pl.ds(base, C)], a_v)
        return base

    for k in range(EPT // C):
        load_chunk(k)

        def sbody(i, _):
            sg = seg_v[pl.ds(i * 16, 16)]
            for h in range(4):
                e = jnp.exp(a_v[arow0 + h, pl.ds(i * 16, 16)])
                plsc.addupdate_scatter(table_v, [sg + (h * NPAD)], e)
            return 0
        lax.fori_loop(0, C // 16, sbody, 0)

    pltpu.sync_copy(table_v, sp_tabs.at[pl.ds(pl.multiple_of(s * TBL, 128), TBL)])
    plsc.subcore_barrier()
    pltpu.sync_copy(sp_tabs.at[pl.ds(pl.multiple_of(s * SLICE, 128), SLICE)],
                    acc_v)
    for t in range(1, NT):
        pltpu.sync_copy(
            sp_tabs.at[pl.ds(pl.multiple_of(t * TBL + s * SLICE, 128), SLICE)],
            tmp_v)

        def mb(i, _):
            acc_v[pl.ds(i * 16, 16)] = (
                acc_v[pl.ds(i * 16, 16)] + tmp_v[pl.ds(i * 16, 16)])
            return 0
        lax.fori_loop(0, SLICE // 16, mb, 0)
    pltpu.sync_copy(acc_v, sp_fin.at[pl.ds(pl.multiple_of(s * SLICE, 128), SLICE)])
    plsc.subcore_barrier()
    pltpu.sync_copy(sp_fin, table_v)

    for k in range(EPT // C):
        base = load_chunk(k)

        def gbody(i, _):
            sg = seg_v[pl.ds(i * 16, 16)]
            for h in range(4):
                e = jnp.exp(a_v[arow0 + h, pl.ds(i * 16, 16)])
                ssum = plsc.load_gather(table_v, [sg + (h * NPAD)])
                w_v[h, pl.ds(i * 16, 16)] = e / (ssum + 1e-16)
            return 0
        lax.fori_loop(0, C // 16, gbody, 0)

        @pl.when(c == 0)
        def _():
            pltpu.sync_copy(w_v, wTi_hbm.at[:, pl.ds(base, C)])

        @pl.when(c == 1)
        def _():
            pltpu.sync_copy(w_v, wTo_hbm.at[:, pl.ds(base, C)])


@functools.lru_cache(maxsize=1)
def _make_softmax_sc():
    mesh = plsc.VectorSubcoreMesh(core_axis_name="c", subcore_axis_name="s")
    return pl.kernel(
        _softmax_sc_body,
        mesh=mesh,
        compiler_params=pltpu.CompilerParams(needs_layout_passes=False),
        out_type=[jax.ShapeDtypeStruct((4, EPAD), jnp.float32),
                  jax.ShapeDtypeStruct((4, EPAD), jnp.float32)],
        scratch_types=[
            pltpu.VMEM((TBL,), jnp.float32),
            pltpu.VMEM((C,), jnp.int32),
            pltpu.VMEM((8, C), jnp.float32),
            pltpu.VMEM((4, C), jnp.float32),
            pltpu.VMEM((SLICE,), jnp.float32),
            pltpu.VMEM((SLICE,), jnp.float32),
            pltpu.VMEM_SHARED((NT * TBL,), jnp.float32),
            pltpu.VMEM_SHARED((TBL,), jnp.float32),
        ],
    )


def _rep8():
    return (jnp.arange(256)[None, :] // 32 == jnp.arange(8)[:, None]).astype(jnp.float32)


def kernel(edge_attr, edge_index, num_nodes,
           W_v_in, b_v_in, W_q_in, b_q_in, W_k_in, b_k_in,
           W_v_out, b_v_out, W_q_out, b_q_out, W_k_out, b_k_out, W_o, b_o):
    del num_nodes
    rep = _rep8()
    rep4 = rep[:4, :128]
    wqk = jnp.concatenate([W_q_in, W_k_in, W_q_out, W_k_out], axis=1)
    bqk = jnp.concatenate([b_q_in, b_k_in, b_q_out, b_k_out])[None, :]
    wv = jnp.concatenate([W_v_in, W_v_out], axis=1)
    bv = jnp.concatenate([b_v_in, b_v_out])[None, :]
    w13 = jnp.concatenate([W_o[0:128], W_o[256:384]], axis=0)
    w24 = jnp.concatenate([W_o[128:256], W_o[384:512]], axis=0)

    nblk = E // BM1
    aT = pl.pallas_call(
        _p1_body,
        grid=(EPAD // BM1,),
        in_specs=[
            pl.BlockSpec((BM1, 128), lambda i: (jnp.minimum(i, nblk - 1), 0)),
            pl.BlockSpec((128, 512), lambda i: (0, 0)),
            pl.BlockSpec((1, 512), lambda i: (0, 0)),
            pl.BlockSpec((8, 256), lambda i: (0, 0)),
        ],
        out_specs=pl.BlockSpec((8, BM1), lambda i: (0, i)),
        out_shape=jax.ShapeDtypeStruct((8, EPAD), jnp.float32),
    )(edge_attr, wqk, bqk, rep)

    pad = jnp.full((EPAD - E,), DUMMY, jnp.int32)
    seg_all = jnp.concatenate(
        [edge_index[0], pad, edge_index[1], pad])

    wT_in, wT_out = _make_softmax_sc()(aT, seg_all)

    out = pl.pallas_call(
        _p2_body,
        grid=(E // BM2,),
        in_specs=[
            pl.BlockSpec((BM2, 128), lambda i: (i, 0)),
            pl.BlockSpec((128, 256), lambda i: (0, 0)),
            pl.BlockSpec((1, 256), lambda i: (0, 0)),
            pl.BlockSpec((4, 128), lambda i: (0, 0)),
            pl.BlockSpec((4, BM2), lambda i: (0, i)),
            pl.BlockSpec((4, BM2), lambda i: (0, i)),
            pl.BlockSpec((256, 128), lambda i: (0, 0)),
            pl.BlockSpec((256, 128), lambda i: (0, 0)),
            pl.BlockSpec((1, 128), lambda i: (0, 0)),
        ],
        out_specs=pl.BlockSpec((BM2, 128), lambda i: (i, 0)),
        out_shape=jax.ShapeDtypeStruct((E, 128), jnp.float32),
    )(edge_attr, wv, bv, rep4, wT_in, wT_out, w13, w24, b_o[None, :])
    return out


# speedup vs baseline: 9.1223x; 1.1404x over previous
import functools
import math

import jax
import jax.numpy as jnp
from jax import lax
from jax.experimental import pallas as pl
from jax.experimental.pallas import tpu as pltpu
from jax.experimental.pallas import tpu_sc as plsc

N_NODES = 10000
E = 320000
D_HEAD = 32
INV_SCALE = 1.0 / math.sqrt(D_HEAD)

NPAD = 10240
DUMMY = 10200
TBL = 4 * NPAD
SLICE = TBL // 16
NT = 16
EPAD = 327680
EPT = EPAD // NT
C = 2560

BM1 = 1280
BM2 = 640


def _p1_body(x_ref, wqk_ref, bqk_ref, rep_ref, aT_ref):
    x = x_ref[...]
    qk = jnp.dot(x, wqk_ref[...], preferred_element_type=jnp.float32) + bqk_ref[...]
    r_in = qk[:, 0:128] * qk[:, 128:256]
    r_out = qk[:, 256:384] * qk[:, 384:512]
    r_all = jnp.concatenate([r_in, r_out], axis=1)
    aT = lax.dot_general(rep_ref[...], r_all, (((1,), (1,)), ((), ())),
                         preferred_element_type=jnp.float32)
    aT_ref[...] = aT * INV_SCALE


def _p2_body(x_ref, wv_ref, bv_ref, rep_ref, wTi_ref, wTo_ref, w13_ref, w24_ref,
             bo_ref, out_ref):
    x = x_ref[...]
    v = jnp.dot(x, wv_ref[...], preferred_element_type=jnp.float32) + bv_ref[...]
    rep4 = rep_ref[...]
    wbc_in = lax.dot_general(wTi_ref[...], rep4, (((0,), (0,)), ((), ())),
                             preferred_element_type=jnp.float32)
    wbc_out = lax.dot_general(wTo_ref[...], rep4, (((0,), (0,)), ((), ())),
                              preferred_element_type=jnp.float32)
    o = v * jnp.concatenate([wbc_in, wbc_out], axis=1)
    z = jnp.dot(o, w24_ref[...], preferred_element_type=jnp.float32)
    up = jnp.roll(z, -1, axis=0)
    dn = jnp.roll(z, 1, axis=0)
    row = lax.broadcasted_iota(jnp.int32, (BM2, 1), 0)
    zsw = jnp.where((row % 2) == 0, up, dn)
    out_ref[...] = (
        jnp.dot(o, w13_ref[...], preferred_element_type=jnp.float32)
        + zsw + bo_ref[...])


def _softmax_sc_body(aT_hbm, seg_hbm, wTi_hbm, wTo_hbm,
                     table_v, seg_v, a_v, w_v, tmp_v, acc_v, sp_tabs, sp_fin):
    c = lax.axis_index("c")
    s = lax.axis_index("s")
    seg_off0 = (1 - c) * EPAD
    arow0 = c * 4
    nch = EPT // C

    zero16 = jnp.zeros((16,), jnp.float32)

    @plsc.parallel_loop(0, TBL, step=16, unroll=8)
    def _(i):
        table_v[pl.ds(i, 16)] = zero16

    def load_chunk(k):
        base = pl.multiple_of(s * EPT + k * C, 128)
        pltpu.sync_copy(seg_hbm.at[pl.ds(pl.multiple_of(seg_off0 + base, 128), C)],
                        seg_v)
        pltpu.sync_copy(aT_hbm.at[:, pl.ds(base, C)], a_v)
        return base

    def sk(k, _):
        load_chunk(k)

        @plsc.parallel_loop(0, C, step=16, unroll=4)
        def _(i):
            sg = seg_v[pl.ds(i, 16)]
            for h in range(4):
                e = jnp.exp(a_v[arow0 + h, pl.ds(i, 16)])
                plsc.addupdate_scatter(table_v, [sg + (h * NPAD)], e)
        return 0
    lax.fori_loop(0, nch, sk, 0)

    pltpu.sync_copy(table_v, sp_tabs.at[pl.ds(pl.multiple_of(s * TBL, 128), TBL)])
    plsc.subcore_barrier()
    pltpu.sync_copy(sp_tabs.at[pl.ds(pl.multiple_of(s * SLICE, 128), SLICE)],
                    acc_v)
    for t in range(1, NT):
        pltpu.sync_copy(
            sp_tabs.at[pl.ds(pl.multiple_of(t * TBL + s * SLICE, 128), SLICE)],
            tmp_v)

        @plsc.parallel_loop(0, SLICE, step=16, unroll=4)
        def _(i):
            acc_v[pl.ds(i, 16)] = acc_v[pl.ds(i, 16)] + tmp_v[pl.ds(i, 16)]
    pltpu.sync_copy(acc_v, sp_fin.at[pl.ds(pl.multiple_of(s * SLICE, 128), SLICE)])
    plsc.subcore_barrier()
    pltpu.sync_copy(sp_fin, table_v)

    def gk(k, _):
        base = load_chunk(k)

        @plsc.parallel_loop(0, C, step=16, unroll=4)
        def _(i):
            sg = seg_v[pl.ds(i, 16)]
            for h in range(4):
                e = jnp.exp(a_v[arow0 + h, pl.ds(i, 16)])
                ssum = plsc.load_gather(table_v, [sg + (h * NPAD)])
                w_v[h, pl.ds(i, 16)] = e / (ssum + 1e-16)

        @pl.when(c == 0)
        def _():
            pltpu.sync_copy(w_v, wTi_hbm.at[:, pl.ds(base, C)])

        @pl.when(c == 1)
        def _():
            pltpu.sync_copy(w_v, wTo_hbm.at[:, pl.ds(base, C)])
        return 0
    lax.fori_loop(0, nch, gk, 0)


@functools.lru_cache(maxsize=1)
def _make_softmax_sc():
    mesh = plsc.VectorSubcoreMesh(core_axis_name="c", subcore_axis_name="s")
    return pl.kernel(
        _softmax_sc_body,
        mesh=mesh,
        compiler_params=pltpu.CompilerParams(needs_layout_passes=False),
        out_type=[jax.ShapeDtypeStruct((4, EPAD), jnp.float32),
                  jax.ShapeDtypeStruct((4, EPAD), jnp.float32)],
        scratch_types=[
            pltpu.VMEM((TBL,), jnp.float32),
            pltpu.VMEM((C,), jnp.int32),
            pltpu.VMEM((8, C), jnp.float32),
            pltpu.VMEM((4, C), jnp.float32),
            pltpu.VMEM((SLICE,), jnp.float32),
            pltpu.VMEM((SLICE,), jnp.float32),
            pltpu.VMEM_SHARED((NT * TBL,), jnp.float32),
            pltpu.VMEM_SHARED((TBL,), jnp.float32),
        ],
    )


def _rep8():
    return (jnp.arange(256)[None, :] // 32 == jnp.arange(8)[:, None]).astype(jnp.float32)


def kernel(edge_attr, edge_index, num_nodes,
           W_v_in, b_v_in, W_q_in, b_q_in, W_k_in, b_k_in,
           W_v_out, b_v_out, W_q_out, b_q_out, W_k_out, b_k_out, W_o, b_o):
    del num_nodes
    rep = _rep8()
    rep4 = rep[:4, :128]
    wqk = jnp.concatenate([W_q_in, W_k_in, W_q_out, W_k_out], axis=1)
    bqk = jnp.concatenate([b_q_in, b_k_in, b_q_out, b_k_out])[None, :]
    wv = jnp.concatenate([W_v_in, W_v_out], axis=1)
    bv = jnp.concatenate([b_v_in, b_v_out])[None, :]
    w13 = jnp.concatenate([W_o[0:128], W_o[256:384]], axis=0)
    w24 = jnp.concatenate([W_o[128:256], W_o[384:512]], axis=0)

    nblk = E // BM1
    aT = pl.pallas_call(
        _p1_body,
        grid=(EPAD // BM1,),
        in_specs=[
            pl.BlockSpec((BM1, 128), lambda i: (jnp.minimum(i, nblk - 1), 0)),
            pl.BlockSpec((128, 512), lambda i: (0, 0)),
            pl.BlockSpec((1, 512), lambda i: (0, 0)),
            pl.BlockSpec((8, 256), lambda i: (0, 0)),
        ],
        out_specs=pl.BlockSpec((8, BM1), lambda i: (0, i)),
        out_shape=jax.ShapeDtypeStruct((8, EPAD), jnp.float32),
    )(edge_attr, wqk, bqk, rep)

    pad = jnp.full((EPAD - E,), DUMMY, jnp.int32)
    seg_all = jnp.concatenate(
        [edge_index[0], pad, edge_index[1], pad])

    wT_in, wT_out = _make_softmax_sc()(aT, seg_all)

    out = pl.pallas_call(
        _p2_body,
        grid=(E // BM2,),
        in_specs=[
            pl.BlockSpec((BM2, 128), lambda i: (i, 0)),
            pl.BlockSpec((128, 256), lambda i: (0, 0)),
            pl.BlockSpec((1, 256), lambda i: (0, 0)),
            pl.BlockSpec((4, 128), lambda i: (0, 0)),
            pl.BlockSpec((4, BM2), lambda i: (0, i)),
            pl.BlockSpec((4, BM2), lambda i: (0, i)),
            pl.BlockSpec((256, 128), lambda i: (0, 0)),
            pl.BlockSpec((256, 128), lambda i: (0, 0)),
            pl.BlockSpec((1, 128), lambda i: (0, 0)),
        ],
        out_specs=pl.BlockSpec((BM2, 128), lambda i: (i, 0)),
        out_shape=jax.ShapeDtypeStruct((E, 128), jnp.float32),
    )(edge_attr, wv, bv, rep4, wT_in, wT_out, w13, w24, b_o[None, :])
    return out


# speedup vs baseline: 12.5200x; 1.3725x over previous
import functools
import math

import jax
import jax.numpy as jnp
from jax import lax
from jax.experimental import pallas as pl
from jax.experimental.pallas import tpu as pltpu
from jax.experimental.pallas import tpu_sc as plsc

N_NODES = 10000
E = 320000
D_HEAD = 32
INV_SCALE = 1.0 / math.sqrt(D_HEAD)

NPAD = 10240
DUMMY = 10200
TBL = 4 * NPAD
SLICE = TBL // 16
NT = 16
EPAD = 327680
EPT = EPAD // NT
C = 2560

BM1 = 2560
BM2 = 1280


def _p1_body(x_ref, wqk_ref, bqk_ref, rep_ref, aT_ref):
    x = x_ref[...]
    qk = jnp.dot(x, wqk_ref[...], preferred_element_type=jnp.float32) + bqk_ref[...]
    r_in = qk[:, 0:128] * qk[:, 128:256]
    r_out = qk[:, 256:384] * qk[:, 384:512]
    r_all = jnp.concatenate([r_in, r_out], axis=1)
    aT = lax.dot_general(rep_ref[...], r_all, (((1,), (1,)), ((), ())),
                         preferred_element_type=jnp.float32)
    aT_ref[...] = aT * INV_SCALE


def _p2_body(x_ref, wv_ref, bv_ref, rep_ref, wTi_ref, wTo_ref, w13_ref, w24_ref,
             bo_ref, out_ref):
    x = x_ref[...]
    v = jnp.dot(x, wv_ref[...], preferred_element_type=jnp.float32) + bv_ref[...]
    rep4 = rep_ref[...]
    wbc_in = lax.dot_general(wTi_ref[...], rep4, (((0,), (0,)), ((), ())),
                             preferred_element_type=jnp.float32)
    wbc_out = lax.dot_general(wTo_ref[...], rep4, (((0,), (0,)), ((), ())),
                              preferred_element_type=jnp.float32)
    o = v * jnp.concatenate([wbc_in, wbc_out], axis=1)
    z = jnp.dot(o, w24_ref[...], preferred_element_type=jnp.float32)
    up = jnp.roll(z, -1, axis=0)
    dn = jnp.roll(z, 1, axis=0)
    row = lax.broadcasted_iota(jnp.int32, (BM2, 1), 0)
    zsw = jnp.where((row % 2) == 0, up, dn)
    out_ref[...] = (
        jnp.dot(o, w13_ref[...], preferred_element_type=jnp.float32)
        + zsw + bo_ref[...])


def _softmax_sc_body(aT_hbm, seg_hbm, wTi_hbm, wTo_hbm,
                     table_v, seg_v, a_v, w_v, tmp_v, acc_v, sp_tabs, sp_fin):
    c = lax.axis_index("c")
    s = lax.axis_index("s")
    seg_off0 = (1 - c) * EPAD
    arow0 = c * 4
    nch = EPT // C

    zero16 = jnp.zeros((16,), jnp.float32)

    @plsc.parallel_loop(0, TBL, step=16, unroll=8)
    def _(i):
        table_v[pl.ds(i, 16)] = zero16

    def load_chunk(k):
        base = pl.multiple_of(s * EPT + k * C, 128)
        pltpu.sync_copy(seg_hbm.at[pl.ds(pl.multiple_of(seg_off0 + base, 128), C)],
                        seg_v)
        pltpu.sync_copy(aT_hbm.at[:, pl.ds(base, C)], a_v)
        return base

    def sk(k, _):
        load_chunk(k)

        @plsc.parallel_loop(0, C, step=16, unroll=4)
        def _(i):
            sg = seg_v[pl.ds(i, 16)]
            for h in range(4):
                e = jnp.exp(a_v[arow0 + h, pl.ds(i, 16)])
                plsc.addupdate_scatter(table_v, [sg + (h * NPAD)], e)
        return 0
    lax.fori_loop(0, nch, sk, 0)

    pltpu.sync_copy(table_v, sp_tabs.at[pl.ds(pl.multiple_of(s * TBL, 128), TBL)])
    plsc.subcore_barrier()
    pltpu.sync_copy(sp_tabs.at[pl.ds(pl.multiple_of(s * SLICE, 128), SLICE)],
                    acc_v)
    for t in range(1, NT):
        pltpu.sync_copy(
            sp_tabs.at[pl.ds(pl.multiple_of(t * TBL + s * SLICE, 128), SLICE)],
            tmp_v)

        @plsc.parallel_loop(0, SLICE, step=16, unroll=4)
        def _(i):
            acc_v[pl.ds(i, 16)] = acc_v[pl.ds(i, 16)] + tmp_v[pl.ds(i, 16)]
    pltpu.sync_copy(acc_v, sp_fin.at[pl.ds(pl.multiple_of(s * SLICE, 128), SLICE)])
    plsc.subcore_barrier()
    pltpu.sync_copy(sp_fin, table_v)

    def gk(k, _):
        base = load_chunk(k)

        @plsc.parallel_loop(0, C, step=16, unroll=4)
        def _(i):
            sg = seg_v[pl.ds(i, 16)]
            for h in range(4):
                e = jnp.exp(a_v[arow0 + h, pl.ds(i, 16)])
                ssum = plsc.load_gather(table_v, [sg + (h * NPAD)])
                w_v[h, pl.ds(i, 16)] = e / (ssum + 1e-16)

        @pl.when(c == 0)
        def _():
            pltpu.sync_copy(w_v, wTi_hbm.at[:, pl.ds(base, C)])

        @pl.when(c == 1)
        def _():
            pltpu.sync_copy(w_v, wTo_hbm.at[:, pl.ds(base, C)])
        return 0
    lax.fori_loop(0, nch, gk, 0)


@functools.lru_cache(maxsize=1)
def _make_softmax_sc():
    mesh = plsc.VectorSubcoreMesh(core_axis_name="c", subcore_axis_name="s")
    return pl.kernel(
        _softmax_sc_body,
        mesh=mesh,
        compiler_params=pltpu.CompilerParams(needs_layout_passes=False),
        out_type=[jax.ShapeDtypeStruct((4, EPAD), jnp.float32),
                  jax.ShapeDtypeStruct((4, EPAD), jnp.float32)],
        scratch_types=[
            pltpu.VMEM((TBL,), jnp.float32),
            pltpu.VMEM((C,), jnp.int32),
            pltpu.VMEM((8, C), jnp.float32),
            pltpu.VMEM((4, C), jnp.float32),
            pltpu.VMEM((SLICE,), jnp.float32),
            pltpu.VMEM((SLICE,), jnp.float32),
            pltpu.VMEM_SHARED((NT * TBL,), jnp.float32),
            pltpu.VMEM_SHARED((TBL,), jnp.float32),
        ],
    )


def _rep8():
    return (jnp.arange(256)[None, :] // 32 == jnp.arange(8)[:, None]).astype(jnp.float32)


def kernel(edge_attr, edge_index, num_nodes,
           W_v_in, b_v_in, W_q_in, b_q_in, W_k_in, b_k_in,
           W_v_out, b_v_out, W_q_out, b_q_out, W_k_out, b_k_out, W_o, b_o):
    del num_nodes
    rep = _rep8()
    rep4 = rep[:4, :128]
    wqk = jnp.concatenate([W_q_in, W_k_in, W_q_out, W_k_out], axis=1)
    bqk = jnp.concatenate([b_q_in, b_k_in, b_q_out, b_k_out])[None, :]
    wv = jnp.concatenate([W_v_in, W_v_out], axis=1)
    bv = jnp.concatenate([b_v_in, b_v_out])[None, :]
    w13 = jnp.concatenate([W_o[0:128], W_o[256:384]], axis=0)
    w24 = jnp.concatenate([W_o[128:256], W_o[384:512]], axis=0)

    nblk = E // BM1
    aT = pl.pallas_call(
        _p1_body,
        grid=(EPAD // BM1,),
        in_specs=[
            pl.BlockSpec((BM1, 128), lambda i: (jnp.minimum(i, nblk - 1), 0)),
            pl.BlockSpec((128, 512), lambda i: (0, 0)),
            pl.BlockSpec((1, 512), lambda i: (0, 0)),
            pl.BlockSpec((8, 256), lambda i: (0, 0)),
        ],
        out_specs=pl.BlockSpec((8, BM1), lambda i: (0, i)),
        out_shape=jax.ShapeDtypeStruct((8, EPAD), jnp.float32),
    )(edge_attr, wqk, bqk, rep)

    pad = jnp.full((EPAD - E,), DUMMY, jnp.int32)
    seg_all = jnp.concatenate(
        [edge_index[0], pad, edge_index[1], pad])

    wT_in, wT_out = _make_softmax_sc()(aT, seg_all)

    out = pl.pallas_call(
        _p2_body,
        grid=(E // BM2,),
        in_specs=[
            pl.BlockSpec((BM2, 128), lambda i: (i, 0)),
            pl.BlockSpec((128, 256), lambda i: (0, 0)),
            pl.BlockSpec((1, 256), lambda i: (0, 0)),
            pl.BlockSpec((4, 128), lambda i: (0, 0)),
            pl.BlockSpec((4, BM2), lambda i: (0, i)),
            pl.BlockSpec((4, BM2), lambda i: (0, i)),
            pl.BlockSpec((256, 128), lambda i: (0, 0)),
            pl.BlockSpec((256, 128), lambda i: (0, 0)),
            pl.BlockSpec((1, 128), lambda i: (0, 0)),
        ],
        out_specs=pl.BlockSpec((BM2, 128), lambda i: (i, 0)),
        out_shape=jax.ShapeDtypeStruct((E, 128), jnp.float32),
    )(edge_attr, wv, bv, rep4, wT_in, wT_out, w13, w24, b_o[None, :])
    return out
